# baseline (speedup 1.0000x reference)
import jax
import jax.numpy as jnp
from jax.experimental import pallas as pl


def _mplayer_kernel(sem_ref, w_ref, adj_ref, out_ref):
    s0 = sem_ref[...]
    w = w_ref[...]
    adj = adj_ref[...]
    h_t = jax.lax.dot_general(
        w, s0, (((0,), (1,)), ((), ())), preferred_element_type=jnp.float32
    )
    sum_t = jax.lax.dot_general(
        h_t, adj, (((1,), (0,)), ((), ())), preferred_element_type=jnp.float32
    )
    counts = jnp.sum(adj, axis=0)
    mean_t = sum_t / jnp.maximum(counts, 1.0)[None, :]
    inv_sqrt2 = 0.7071067811865476
    gelu_t = 0.5 * mean_t * (1.0 + jax.lax.erf(mean_t * inv_sqrt2))
    out_ref[...] = gelu_t.T


def kernel(adj, semantics, attention_masks, W):
    n, seq, hidden = semantics.shape
    sem_flat = semantics.reshape(n, seq * hidden)
    return pl.pallas_call(
        _mplayer_kernel,
        grid=(1,),
        in_specs=[
            pl.BlockSpec((n, hidden), lambda i: (0, 0)),
            pl.BlockSpec((hidden, hidden), lambda i: (0, 0)),
            pl.BlockSpec((n, n), lambda i: (0, 0)),
        ],
        out_specs=pl.BlockSpec((n, hidden), lambda i: (0, 0)),
        out_shape=jax.ShapeDtypeStruct((n, hidden), jnp.float32),
    )(sem_flat, W, adj)


# speedup vs baseline: 4.8792x; 4.8792x over previous
import jax
import jax.numpy as jnp
from jax.experimental import pallas as pl


def _mplayer_kernel(s0_ref, w_ref, adj_ref, out_ref):
    s0 = s0_ref[...]
    w = w_ref[...]
    adj = adj_ref[...]
    h_t = jax.lax.dot_general(
        w, s0, (((0,), (1,)), ((), ())), preferred_element_type=jnp.float32
    )
    sum_t = jax.lax.dot_general(
        h_t, adj, (((1,), (0,)), ((), ())), preferred_element_type=jnp.float32
    )
    counts = jnp.sum(adj, axis=0)
    mean_t = sum_t / jnp.maximum(counts, 1.0)[None, :]
    inv_sqrt2 = 0.7071067811865476
    gelu_t = 0.5 * mean_t * (1.0 + jax.lax.erf(mean_t * inv_sqrt2))
    out_ref[...] = gelu_t.T


def kernel(adj, semantics, attention_masks, W):
    n, seq, hidden = semantics.shape
    s0 = semantics[:, 0, :]
    return pl.pallas_call(
        _mplayer_kernel,
        out_shape=jax.ShapeDtypeStruct((n, hidden), jnp.float32),
    )(s0, W, adj)


# speedup vs baseline: 4.9513x; 1.0148x over previous
import jax
import jax.numpy as jnp
from jax.experimental import pallas as pl
from jax.experimental.pallas import tpu as pltpu

_BN = 512


def _mplayer_kernel(s0_ref, w_ref, adj_ref, out_ref, ht_ref):
    @pl.when(pl.program_id(0) == 0)
    def _():
        ht_ref[...] = jax.lax.dot_general(
            w_ref[...], s0_ref[...], (((0,), (1,)), ((), ())),
            preferred_element_type=jnp.float32,
        )

    adj = adj_ref[...]
    sum_t = jax.lax.dot_general(
        ht_ref[...], adj, (((1,), (0,)), ((), ())),
        preferred_element_type=jnp.float32,
    )
    counts = jnp.sum(adj, axis=0)
    mean_t = sum_t / jnp.maximum(counts, 1.0)[None, :]
    inv_sqrt2 = 0.7071067811865476
    gelu_t = 0.5 * mean_t * (1.0 + jax.lax.erf(mean_t * inv_sqrt2))
    out_ref[...] = gelu_t.T


def kernel(adj, semantics, attention_masks, W):
    n = adj.shape[0]
    hidden = W.shape[0]
    s0 = semantics[:, 0, :]
    return pl.pallas_call(
        _mplayer_kernel,
        grid=(n // _BN,),
        in_specs=[
            pl.BlockSpec((n, hidden), lambda j: (0, 0)),
            pl.BlockSpec((hidden, hidden), lambda j: (0, 0)),
            pl.BlockSpec((n, _BN), lambda j: (0, j)),
        ],
        out_specs=pl.BlockSpec((_BN, hidden), lambda j: (j, 0)),
        out_shape=jax.ShapeDtypeStruct((n, hidden), jnp.float32),
        scratch_shapes=[pltpu.VMEM((hidden, n), jnp.float32)],
    )(s0, W, adj)
